# baseline (speedup 1.0000x reference)
import dataclasses
import functools

import jax
import jax.numpy as jnp
from jax import lax
from jax.experimental import pallas as pl
from jax.experimental.pallas import tpu as pltpu
from jax.experimental.pallas import tpu_sc as plsc

_B, _C, _H, _W = 4, 96, 384, 384
_BC = 2
_NCHUNK = _B // _BC
_CPIX = _BC * _H * _W
_NC, _NS, _LANES = 2, 16, 16
_NW = _NC * _NS
_PPW = _CPIX // _NW
_GP = 64
_NWIN = _PPW // _GP
_NBUF = 3
_HB = 8
_ROWS_BLK = _HB * _W


def _chunk_prep_body(img_ref, flow_ref, rows_ref, idxq_ref, wts_ref):
    b = pl.program_id(0)
    i = pl.program_id(1)
    x = img_ref[0].astype(jnp.bfloat16).reshape(_C, _ROWS_BLK)
    rows_ref[...] = jnp.swapaxes(x, 0, 1)
    fx = flow_ref[0, 0]
    fy = flow_ref[0, 1]
    xw = lax.broadcasted_iota(jnp.int32, (_HB, _W), 1).astype(jnp.float32)
    yh = (lax.broadcasted_iota(jnp.int32, (_HB, _W), 0)
          + i * _HB).astype(jnp.float32)
    xs = jnp.clip(xw + fx, 0.0, float(_W - 1))
    ys = jnp.clip(yh + fy, 0.0, float(_H - 1))
    x0 = jnp.minimum(jnp.floor(xs), float(_W - 2))
    y0 = jnp.minimum(jnp.floor(ys), float(_H - 2))
    wx1 = xs - x0
    wx0 = 1.0 - wx1
    wy1 = ys - y0
    wy0 = 1.0 - wy1
    x0i = x0.astype(jnp.int32)
    y0i = y0.astype(jnp.int32)
    q0 = (b * _H + y0i) * _W + x0i
    idxq_ref[0, 0] = q0
    idxq_ref[0, 1] = q0 + 1
    idxq_ref[0, 2] = q0 + _W
    idxq_ref[0, 3] = q0 + _W + 1
    wts_ref[0, 0] = wy0 * wx0
    wts_ref[0, 1] = wy0 * wx1
    wts_ref[0, 2] = wy1 * wx0
    wts_ref[0, 3] = wy1 * wx1


def _chunk_prep(img, flow, b0):
    nhb = _H // _HB
    return pl.pallas_call(
        _chunk_prep_body,
        grid=(_BC, nhb),
        in_specs=[
            pl.BlockSpec((1, _C, _HB, _W), lambda b, i: (b0 + b, 0, i, 0)),
            pl.BlockSpec((1, 2, _HB, _W), lambda b, i: (b0 + b, 0, i, 0)),
        ],
        out_specs=[
            pl.BlockSpec((_ROWS_BLK, _C), lambda b, i: (b * nhb + i, 0)),
            pl.BlockSpec((1, 4, _HB, _W), lambda b, i: (b, 0, i, 0)),
            pl.BlockSpec((1, 4, _HB, _W), lambda b, i: (b, 0, i, 0)),
        ],
        out_shape=[
            jax.ShapeDtypeStruct((_CPIX, _C), jnp.bfloat16),
            jax.ShapeDtypeStruct((_BC, 4, _H, _W), jnp.int32),
            jax.ShapeDtypeStruct((_BC, 4, _H, _W), jnp.float32),
        ],
    )(img, flow)


def _from_rows2_body(r01_ref, r23_ref, out_ref):
    b = pl.program_id(0)
    eye = jnp.eye(_C, dtype=jnp.bfloat16)

    def emit(src_ref):
        for h in range(_HB):
            blk = src_ref[pl.ds(h * _W, _W), :]
            t = jax.lax.dot_general(eye, blk, (((0,), (1,)), ((), ())),
                                    preferred_element_type=jnp.float32)
            out_ref[0, :, h, :] = t

    @pl.when(b < _BC)
    def _():
        emit(r01_ref)

    @pl.when(b >= _BC)
    def _():
        emit(r23_ref)


def _from_rows2(rows01, rows23):
    nhb = _H // _HB
    return pl.pallas_call(
        _from_rows2_body,
        grid=(_B, nhb),
        in_specs=[
            pl.BlockSpec((_ROWS_BLK, _C),
                         lambda b, i: (jnp.minimum(b, _BC - 1) * nhb + i, 0)),
            pl.BlockSpec((_ROWS_BLK, _C),
                         lambda b, i: (jnp.maximum(b - _BC, 0) * nhb + i, 0)),
        ],
        out_specs=pl.BlockSpec((1, _C, _HB, _W), lambda b, i: (b, 0, i, 0)),
        out_shape=jax.ShapeDtypeStruct((_B, _C, _H, _W), jnp.float32),
    )(rows01, rows23)


def _sc_warp2(rows01, idxq01, wts01, rows23, idxq23, wts23):
    mesh = plsc.VectorSubcoreMesh(core_axis_name="c", subcore_axis_name="s")
    cp = pltpu.CompilerParams()
    for f, v in (("needs_layout_passes", False), ("use_tc_tiling_on_sc", False)):
        if f in pltpu.CompilerParams.__dataclass_fields__:
            cp = dataclasses.replace(cp, **{f: v})

    nbuf = _NBUF

    @functools.partial(
        pl.kernel,
        mesh=mesh,
        compiler_params=cp,
        out_type=[jax.ShapeDtypeStruct((_CPIX, _C), jnp.bfloat16),
                  jax.ShapeDtypeStruct((_CPIX, _C), jnp.bfloat16)],
        scratch_types=[
            pltpu.VMEM((nbuf, 4 * _GP), jnp.int32),
            pltpu.VMEM((nbuf, 4 * _GP), jnp.float32),
            pltpu.VMEM((nbuf, 4 * _GP, _C), jnp.bfloat16),
            pltpu.VMEM((nbuf, _GP, _C), jnp.bfloat16),
            pltpu.SemaphoreType.DMA((nbuf,)),
            pltpu.SemaphoreType.DMA((nbuf,)),
            pltpu.SemaphoreType.DMA((nbuf,)),
        ],
    )
    def warp2_kernel(rows01_hbm, idx01_hbm, wts01_hbm,
                     rows23_hbm, idx23_hbm, wts23_hbm,
                     out01_hbm, out23_hbm,
                     idx_v, w_v, r_v, o_v, sem_ld, sem_g, sem_st):
        for img_hbm, idx_hbm, wts_hbm, out_hbm in (
                (rows01_hbm, idx01_hbm, wts01_hbm, out01_hbm),
                (rows23_hbm, idx23_hbm, wts23_hbm, out23_hbm)):
            _chunk_pipeline(img_hbm, idx_hbm, wts_hbm, out_hbm,
                            idx_v, w_v, r_v, o_v, sem_ld, sem_g, sem_st,
                            nbuf)

    return warp2_kernel(rows01, idxq01, wts01, rows23, idxq23, wts23)


def _chunk_pipeline(img_hbm, idx_hbm, wts_hbm, out_hbm,
                    idx_v, w_v, r_v, o_v, sem_ld, sem_g, sem_st, nbuf):
        wid = lax.axis_index("s") * _NC + lax.axis_index("c")
        base = wid * _PPW
        hw = _H * _W

        def _bhw(win):
            p0 = base + win * _GP
            b = p0 // hw
            rem = p0 - b * hw
            h = rem // _W
            w0 = rem - h * _W
            return b, h, w0

        def issue_load(win, j):
            b, h, w0 = _bhw(win)
            for c in range(4):
                pltpu.async_copy(idx_hbm.at[b, c, h, pl.ds(w0, _GP)],
                                 idx_v.at[j, pl.ds(c * _GP, _GP)], sem_ld.at[j])
                pltpu.async_copy(wts_hbm.at[b, c, h, pl.ds(w0, _GP)],
                                 w_v.at[j, pl.ds(c * _GP, _GP)], sem_ld.at[j])

        def wait_load(win, j):
            b, h, w0 = _bhw(win)
            for c in range(4):
                pltpu.make_async_copy(idx_hbm.at[b, c, h, pl.ds(w0, _GP)],
                                      idx_v.at[j, pl.ds(c * _GP, _GP)],
                                      sem_ld.at[j]).wait()
                pltpu.make_async_copy(wts_hbm.at[b, c, h, pl.ds(w0, _GP)],
                                      w_v.at[j, pl.ds(c * _GP, _GP)],
                                      sem_ld.at[j]).wait()

        def issue_gather(j):
            for c in range(4):
                pltpu.async_copy(img_hbm.at[idx_v.at[j, pl.ds(c * _GP, _GP)]],
                                 r_v.at[j, pl.ds(c * _GP, _GP)], sem_g.at[j])

        def wait_gather(j):
            for c in range(4):
                pltpu.make_async_copy(img_hbm.at[idx_v.at[j, pl.ds(c * _GP, _GP)]],
                                      r_v.at[j, pl.ds(c * _GP, _GP)],
                                      sem_g.at[j]).wait()

        def issue_store(win, j):
            pltpu.async_copy(o_v.at[j], out_hbm.at[pl.ds(base + win * _GP, _GP)],
                             sem_st.at[j])

        def wait_store(win, j):
            pltpu.make_async_copy(o_v.at[j], out_hbm.at[pl.ds(base + win * _GP, _GP)],
                                  sem_st.at[j]).wait()

        def combine(j):
            @pl.loop(0, _GP)
            def _px(g):
                w0 = plsc.load_gather(w_v.at[j], [jnp.full((_LANES,), g, jnp.int32)])
                w1 = plsc.load_gather(w_v.at[j], [jnp.full((_LANES,), _GP + g, jnp.int32)])
                w2 = plsc.load_gather(w_v.at[j], [jnp.full((_LANES,), 2 * _GP + g, jnp.int32)])
                w3 = plsc.load_gather(w_v.at[j], [jnp.full((_LANES,), 3 * _GP + g, jnp.int32)])
                for k in range(_C // (2 * _LANES)):
                    s = pl.ds(k * 2 * _LANES, 2 * _LANES)
                    a0, b0 = plsc.unpack(r_v[j, g, s],
                                         format=plsc.PackFormat.INTERLEAVED)
                    a1, b1 = plsc.unpack(r_v[j, _GP + g, s],
                                         format=plsc.PackFormat.INTERLEAVED)
                    a2, b2 = plsc.unpack(r_v[j, 2 * _GP + g, s],
                                         format=plsc.PackFormat.INTERLEAVED)
                    a3, b3 = plsc.unpack(r_v[j, 3 * _GP + g, s],
                                         format=plsc.PackFormat.INTERLEAVED)
                    oa = w0 * a0 + w1 * a1 + w2 * a2 + w3 * a3
                    ob = w0 * b0 + w1 * b1 + w2 * b2 + w3 * b3
                    o_v[j, g, s] = plsc.pack(oa, ob,
                                             format=plsc.PackFormat.INTERLEAVED)

        issue_load(0, 0)
        wait_load(0, 0)
        issue_gather(0)
        issue_load(1, 1)

        @pl.loop(0, _NWIN // nbuf)
        def _outer(wo):
            for j in range(nbuf):
                w = wo * nbuf + j
                s1 = (j + 1) % nbuf
                s2 = (j + 2) % nbuf

                @pl.when(w + 1 < _NWIN)
                def _():
                    wait_load(w + 1, s1)
                    issue_gather(s1)

                @pl.when(w + 2 < _NWIN)
                def _():
                    issue_load(w + 2, s2)

                wait_gather(j)

                @pl.when(w >= nbuf)
                def _():
                    wait_store(w - nbuf, j)

                combine(j)
                issue_store(w, j)

        for j in range(nbuf):
            wait_store(_NWIN - nbuf + j, (_NWIN - nbuf + j) % nbuf)


def kernel(img, flow):
    rows01, idxq01, wts01 = _chunk_prep(img, flow, 0)
    rows23, idxq23, wts23 = _chunk_prep(img, flow, _BC)
    out01, out23 = _sc_warp2(rows01, idxq01, wts01, rows23, idxq23, wts23)
    return _from_rows2(out01, out23)


# speedup vs baseline: 1.1091x; 1.1091x over previous
import dataclasses
import functools

import jax
import jax.numpy as jnp
from jax import lax
from jax.experimental import pallas as pl
from jax.experimental.pallas import tpu as pltpu
from jax.experimental.pallas import tpu_sc as plsc

_B, _C, _H, _W = 4, 96, 384, 384
_BC = 2
_NCHUNK = _B // _BC
_CPIX = _BC * _H * _W
_NC, _NS, _LANES = 2, 16, 16
_NW = _NC * _NS
_PPW = _CPIX // _NW
_GP = 64
_NWIN = _PPW // _GP
_NBUF = 3
_HB = 8
_ROWS_BLK = _HB * _W


def _chunk_prep_body(img_ref, flow_ref, rows_ref, idxq_ref, wts_ref):
    b = pl.program_id(0)
    i = pl.program_id(1)
    x = img_ref[0].astype(jnp.bfloat16).reshape(_C, _ROWS_BLK)
    rows_ref[...] = jnp.swapaxes(x, 0, 1)
    fx = flow_ref[0, 0]
    fy = flow_ref[0, 1]
    xw = lax.broadcasted_iota(jnp.int32, (_HB, _W), 1).astype(jnp.float32)
    yh = (lax.broadcasted_iota(jnp.int32, (_HB, _W), 0)
          + i * _HB).astype(jnp.float32)
    xs = jnp.clip(xw + fx, 0.0, float(_W - 1))
    ys = jnp.clip(yh + fy, 0.0, float(_H - 1))
    x0 = jnp.minimum(jnp.floor(xs), float(_W - 2))
    y0 = jnp.minimum(jnp.floor(ys), float(_H - 2))
    wx1 = xs - x0
    wx0 = 1.0 - wx1
    wy1 = ys - y0
    wy0 = 1.0 - wy1
    x0i = x0.astype(jnp.int32)
    y0i = y0.astype(jnp.int32)
    q0 = (b * _H + y0i) * _W + x0i
    idxq_ref[0, 0] = q0
    idxq_ref[0, 1] = q0 + 1
    idxq_ref[0, 2] = q0 + _W
    idxq_ref[0, 3] = q0 + _W + 1
    wts_ref[0, 0] = wy0 * wx0
    wts_ref[0, 1] = wy0 * wx1
    wts_ref[0, 2] = wy1 * wx0
    wts_ref[0, 3] = wy1 * wx1


def _chunk_prep(img, flow, b0):
    nhb = _H // _HB
    return pl.pallas_call(
        _chunk_prep_body,
        grid=(_BC, nhb),
        in_specs=[
            pl.BlockSpec((1, _C, _HB, _W), lambda b, i: (b0 + b, 0, i, 0)),
            pl.BlockSpec((1, 2, _HB, _W), lambda b, i: (b0 + b, 0, i, 0)),
        ],
        out_specs=[
            pl.BlockSpec((_ROWS_BLK, _C), lambda b, i: (b * nhb + i, 0)),
            pl.BlockSpec((1, 4, _HB, _W), lambda b, i: (b, 0, i, 0)),
            pl.BlockSpec((1, 4, _HB, _W), lambda b, i: (b, 0, i, 0)),
        ],
        out_shape=[
            jax.ShapeDtypeStruct((_CPIX, _C), jnp.bfloat16),
            jax.ShapeDtypeStruct((_BC, 4, _H, _W), jnp.int32),
            jax.ShapeDtypeStruct((_BC, 4, _H, _W), jnp.float32),
        ],
    )(img, flow)


def _from_rows2_body(r01_ref, r23_ref, out_ref):
    b = pl.program_id(0)
    eye = jnp.eye(_C, dtype=jnp.bfloat16)

    def emit(src_ref):
        for h in range(_HB):
            blk = src_ref[pl.ds(h * _W, _W), :]
            t = jax.lax.dot_general(eye, blk, (((0,), (1,)), ((), ())),
                                    preferred_element_type=jnp.float32)
            out_ref[0, :, h, :] = t

    @pl.when(b < _BC)
    def _():
        emit(r01_ref)

    @pl.when(b >= _BC)
    def _():
        emit(r23_ref)


def _from_rows2(rows01, rows23):
    nhb = _H // _HB
    return pl.pallas_call(
        _from_rows2_body,
        grid=(_B, nhb),
        in_specs=[
            pl.BlockSpec((_ROWS_BLK, _C),
                         lambda b, i: (jnp.minimum(b, _BC - 1) * nhb + i, 0)),
            pl.BlockSpec((_ROWS_BLK, _C),
                         lambda b, i: (jnp.maximum(b - _BC, 0) * nhb + i, 0)),
        ],
        out_specs=pl.BlockSpec((1, _C, _HB, _W), lambda b, i: (b, 0, i, 0)),
        out_shape=jax.ShapeDtypeStruct((_B, _C, _H, _W), jnp.float32),
    )(rows01, rows23)


def _sc_warp(rows01, idxq01, wts01):
    mesh = plsc.VectorSubcoreMesh(core_axis_name="c", subcore_axis_name="s")
    cp = pltpu.CompilerParams()
    for f, v in (("needs_layout_passes", False), ("use_tc_tiling_on_sc", False)):
        if f in pltpu.CompilerParams.__dataclass_fields__:
            cp = dataclasses.replace(cp, **{f: v})

    nbuf = _NBUF

    @functools.partial(
        pl.kernel,
        mesh=mesh,
        compiler_params=cp,
        out_type=jax.ShapeDtypeStruct((_CPIX, _C), jnp.bfloat16),
        scratch_types=[
            pltpu.VMEM((nbuf, 4 * _GP), jnp.int32),
            pltpu.VMEM((nbuf, 4 * _GP), jnp.float32),
            pltpu.VMEM((nbuf, 4 * _GP, _C), jnp.bfloat16),
            pltpu.VMEM((nbuf, _GP, _C), jnp.bfloat16),
            pltpu.SemaphoreType.DMA((nbuf,)),
            pltpu.SemaphoreType.DMA((nbuf,)),
            pltpu.SemaphoreType.DMA((nbuf,)),
        ],
    )
    def warp_kernel(img_hbm, idx_hbm, wts_hbm, out_hbm,
                    idx_v, w_v, r_v, o_v, sem_ld, sem_g, sem_st):
        _chunk_pipeline(img_hbm, idx_hbm, wts_hbm, out_hbm,
                        idx_v, w_v, r_v, o_v, sem_ld, sem_g, sem_st, nbuf)

    return warp_kernel(rows01, idxq01, wts01)


def _chunk_pipeline(img_hbm, idx_hbm, wts_hbm, out_hbm,
                    idx_v, w_v, r_v, o_v, sem_ld, sem_g, sem_st, nbuf):
        wid = lax.axis_index("s") * _NC + lax.axis_index("c")
        base = wid * _PPW
        hw = _H * _W

        def _bhw(win):
            p0 = base + win * _GP
            b = p0 // hw
            rem = p0 - b * hw
            h = rem // _W
            w0 = rem - h * _W
            return b, h, w0

        def issue_load(win, j):
            b, h, w0 = _bhw(win)
            for c in range(4):
                pltpu.async_copy(idx_hbm.at[b, c, h, pl.ds(w0, _GP)],
                                 idx_v.at[j, pl.ds(c * _GP, _GP)], sem_ld.at[j])
                pltpu.async_copy(wts_hbm.at[b, c, h, pl.ds(w0, _GP)],
                                 w_v.at[j, pl.ds(c * _GP, _GP)], sem_ld.at[j])

        def wait_load(win, j):
            b, h, w0 = _bhw(win)
            for c in range(4):
                pltpu.make_async_copy(idx_hbm.at[b, c, h, pl.ds(w0, _GP)],
                                      idx_v.at[j, pl.ds(c * _GP, _GP)],
                                      sem_ld.at[j]).wait()
                pltpu.make_async_copy(wts_hbm.at[b, c, h, pl.ds(w0, _GP)],
                                      w_v.at[j, pl.ds(c * _GP, _GP)],
                                      sem_ld.at[j]).wait()

        def issue_gather(j):
            for c in range(4):
                pltpu.async_copy(img_hbm.at[idx_v.at[j, pl.ds(c * _GP, _GP)]],
                                 r_v.at[j, pl.ds(c * _GP, _GP)], sem_g.at[j])

        def wait_gather(j):
            for c in range(4):
                pltpu.make_async_copy(img_hbm.at[idx_v.at[j, pl.ds(c * _GP, _GP)]],
                                      r_v.at[j, pl.ds(c * _GP, _GP)],
                                      sem_g.at[j]).wait()

        def issue_store(win, j):
            pltpu.async_copy(o_v.at[j], out_hbm.at[pl.ds(base + win * _GP, _GP)],
                             sem_st.at[j])

        def wait_store(win, j):
            pltpu.make_async_copy(o_v.at[j], out_hbm.at[pl.ds(base + win * _GP, _GP)],
                                  sem_st.at[j]).wait()

        def combine(j):
            @pl.loop(0, _GP)
            def _px(g):
                w0 = plsc.load_gather(w_v.at[j], [jnp.full((_LANES,), g, jnp.int32)])
                w1 = plsc.load_gather(w_v.at[j], [jnp.full((_LANES,), _GP + g, jnp.int32)])
                w2 = plsc.load_gather(w_v.at[j], [jnp.full((_LANES,), 2 * _GP + g, jnp.int32)])
                w3 = plsc.load_gather(w_v.at[j], [jnp.full((_LANES,), 3 * _GP + g, jnp.int32)])
                for k in range(_C // (2 * _LANES)):
                    s = pl.ds(k * 2 * _LANES, 2 * _LANES)
                    a0, b0 = plsc.unpack(r_v[j, g, s],
                                         format=plsc.PackFormat.INTERLEAVED)
                    a1, b1 = plsc.unpack(r_v[j, _GP + g, s],
                                         format=plsc.PackFormat.INTERLEAVED)
                    a2, b2 = plsc.unpack(r_v[j, 2 * _GP + g, s],
                                         format=plsc.PackFormat.INTERLEAVED)
                    a3, b3 = plsc.unpack(r_v[j, 3 * _GP + g, s],
                                         format=plsc.PackFormat.INTERLEAVED)
                    oa = w0 * a0 + w1 * a1 + w2 * a2 + w3 * a3
                    ob = w0 * b0 + w1 * b1 + w2 * b2 + w3 * b3
                    o_v[j, g, s] = plsc.pack(oa, ob,
                                             format=plsc.PackFormat.INTERLEAVED)

        issue_load(0, 0)
        wait_load(0, 0)
        issue_gather(0)
        issue_load(1, 1)

        @pl.loop(0, _NWIN // nbuf)
        def _outer(wo):
            for j in range(nbuf):
                w = wo * nbuf + j
                s1 = (j + 1) % nbuf
                s2 = (j + 2) % nbuf

                @pl.when(w + 1 < _NWIN)
                def _():
                    wait_load(w + 1, s1)
                    issue_gather(s1)

                @pl.when(w + 2 < _NWIN)
                def _():
                    issue_load(w + 2, s2)

                wait_gather(j)

                @pl.when(w >= nbuf)
                def _():
                    wait_store(w - nbuf, j)

                combine(j)
                issue_store(w, j)

        for j in range(nbuf):
            wait_store(_NWIN - nbuf + j, (_NWIN - nbuf + j) % nbuf)


def kernel(img, flow):
    out_chunks = []
    for k in range(_NCHUNK):
        rows, idxq, wts = _chunk_prep(img, flow, k * _BC)
        out_chunks.append(_sc_warp(rows, idxq, wts))
    return _from_rows2(out_chunks[0], out_chunks[1])
